# baseline (speedup 1.0000x reference)
import functools

import jax
import jax.numpy as jnp
from jax import lax
from jax.experimental import pallas as pl
from jax.experimental.pallas import tpu as pltpu

_N, _J, _K = 4096, 1024, 16
_R = 256


def _onehot_kernel(x_ref, o_ref):
    g = pl.program_id(0)
    xv = x_ref[pl.ds(g * _R, _R), :]
    idx = jnp.floor(jnp.clip(xv, 0.0, 15.5)).astype(jnp.int32)
    ks = lax.broadcasted_iota(jnp.int32, (_R, _K, _J), 1)
    o_ref[...] = (idx[:, None, :] == ks).astype(jnp.float32)


@functools.partial(jax.jit, static_argnames=("interpret",))
def kernel(x, interpret=False):
    out = pl.pallas_call(
        _onehot_kernel,
        grid=(_N // _R,),
        in_specs=[pl.BlockSpec((_N, _J), lambda g: (0, 0))],
        out_specs=pl.BlockSpec((_R, _K, _J), lambda g: (g, 0, 0)),
        out_shape=jax.ShapeDtypeStruct((_N, _K, _J), jnp.float32),
        compiler_params=pltpu.CompilerParams(
            dimension_semantics=("arbitrary",),
        ),
        interpret=interpret,
    )(x)
    return jnp.transpose(out, (0, 2, 1))


# speedup vs baseline: 1.0045x; 1.0045x over previous
import functools

import jax
import jax.numpy as jnp
from jax import lax
from jax.experimental import pallas as pl
from jax.experimental.pallas import tpu as pltpu

_N, _J, _K = 4096, 1024, 16
_R = 256


def _onehot_kernel(x_ref, o_ref):
    xv = x_ref[...]
    idx = jnp.floor(jnp.clip(xv, 0.0, 15.5)).astype(jnp.int32)
    ks = lax.broadcasted_iota(jnp.int32, (_R, _K, _J), 1)
    o_ref[...] = (idx[:, None, :] == ks).astype(jnp.float32)


@functools.partial(jax.jit, static_argnames=("interpret",))
def kernel(x, interpret=False):
    out = pl.pallas_call(
        _onehot_kernel,
        grid=(_N // _R,),
        in_specs=[pl.BlockSpec((_R, _J), lambda g: (g, 0))],
        out_specs=pl.BlockSpec((_R, _K, _J), lambda g: (g, 0, 0)),
        out_shape=jax.ShapeDtypeStruct((_N, _K, _J), jnp.float32),
        compiler_params=pltpu.CompilerParams(
            dimension_semantics=("arbitrary",),
        ),
        interpret=interpret,
    )(x)
    return jnp.transpose(out, (0, 2, 1))
